# baseline (speedup 1.0000x reference)
import functools

import jax
import jax.numpy as jnp
from jax import lax
from jax.experimental import pallas as pl
from jax.experimental.pallas import tpu as pltpu
from jax.experimental.pallas import tpu_sc as plsc

VOCAB = 100000
D_MODEL = 1024
BATCH = 128
Q_LEN = 32

_L = 16
_NVEC = D_MODEL // _L
_B = BATCH * Q_LEN
_NIN = 5
_NOUT = 2
_CH = 16
_NCH = 8
_CBLK = 8
_BROWS = BATCH // 32


def _sc_kernel_call(table, y, scalars, pos):
    info = plsc.get_sparse_core_info()
    nc, ns = info.num_cores, info.num_subcores
    nw = nc * ns
    rows_per_w = _B // nw
    assert rows_per_w == _NCH * _CH

    mesh = plsc.VectorSubcoreMesh(core_axis_name="c", subcore_axis_name="s")

    @functools.partial(
        pl.kernel,
        mesh=mesh,
        out_type=jax.ShapeDtypeStruct((_B, D_MODEL), jnp.float32),
        scratch_types=(
            [pltpu.VMEM((_BROWS, Q_LEN), jnp.int32)]
            + [pltpu.VMEM((_CH, D_MODEL), jnp.float32)
               for _ in range(_NIN + _NOUT)]
            + [pltpu.VMEM((D_MODEL,), jnp.float32),
               pltpu.VMEM((D_MODEL,), jnp.float32),
               pltpu.VMEM((2, _L), jnp.float32)]
            + [pltpu.SemaphoreType.DMA for _ in range(_NIN + 2 * _NOUT + 1)]
        ),
    )
    def k(table_hbm, y_hbm, sc_hbm, pos_hbm, out_hbm, *refs):
        idx_v = refs[0]
        ibufs = refs[1:1 + _NIN]
        obufs = refs[1 + _NIN:1 + _NIN + _NOUT]
        pos_v, spos, scal_v = refs[1 + _NIN + _NOUT:4 + _NIN + _NOUT]
        gsems = refs[4 + _NIN + _NOUT:4 + 2 * _NIN + _NOUT]
        wsems = refs[4 + 2 * _NIN + _NOUT:4 + 2 * _NIN + 3 * _NOUT]
        psem = refs[4 + 2 * _NIN + 3 * _NOUT]

        wid = lax.axis_index("s") * nc + lax.axis_index("c")
        base = wid * rows_per_w

        scal_cp = pltpu.async_copy(sc_hbm, scal_v, psem)
        pos_cp = pltpu.async_copy(pos_hbm, pos_v, psem)
        pltpu.sync_copy(y_hbm.at[pl.ds(wid * _BROWS, _BROWS)], idx_v)

        def start_gather(c, s):
            row, off = (c * _CH) // Q_LEN, (c * _CH) % Q_LEN
            idx = idx_v.at[row, pl.ds(off, _CH)]
            return pltpu.async_copy(table_hbm.at[idx], ibufs[s], gsems[s])

        def start_write_half(c, s, h):
            hr = _CH // 2
            return pltpu.async_copy(
                obufs[s].at[pl.ds(h * hr, hr)],
                out_hbm.at[pl.ds(base + c * _CH + h * hr, hr)],
                wsems[2 * s + h])

        inflight_g = [None] * _NIN
        for c in range(_NIN - 1):
            inflight_g[c] = start_gather(c, c)

        scal_cp.wait()
        pos_cp.wait()

        av = scal_v[0, :]
        sv = scal_v[1, :]

        def scale_pos(j, carry):
            spos[pl.ds(j * _L, _L)] = pos_v[pl.ds(j * _L, _L)] * av
            return carry

        lax.fori_loop(0, _NVEC, scale_pos, 0)

        def compute_half(si, so, h):
            src = ibufs[si]
            dst = obufs[so]
            hr = _CH // 2

            def block_body(b, carry):
                pvs = [spos[pl.ds((b * _CBLK + j) * _L, _L)]
                       for j in range(_CBLK)]

                @plsc.parallel_loop(h * hr, (h + 1) * hr, 1, unroll=2)
                def rows_body(r):
                    for j in range(_CBLK):
                        sl = pl.ds((b * _CBLK + j) * _L, _L)
                        dst[r, sl] = src[r, sl] * sv + pvs[j]

                return carry

            lax.fori_loop(0, _NVEC // _CBLK, block_body, 0)

        inflight_w = [[None, None] for _ in range(_NOUT)]
        for i in range(_NCH):
            si = i % _NIN
            so = i % _NOUT
            j = i + _NIN - 1
            if j < _NCH:
                inflight_g[j % _NIN] = start_gather(j, j % _NIN)
            inflight_g[si].wait()
            for h in range(2):
                if inflight_w[so][h] is not None:
                    inflight_w[so][h].wait()
                    inflight_w[so][h] = None
                compute_half(si, so, h)
                inflight_w[so][h] = start_write_half(i, so, h)
        for s in range(_NOUT):
            for h in range(2):
                if inflight_w[s][h] is not None:
                    inflight_w[s][h].wait()

    return k(table, y, scalars, pos)


def kernel(emb_table, alpha, pe, x_scale, y, idx_plus_len):
    y32 = y.astype(jnp.int32)
    pos = lax.dynamic_index_in_dim(pe[0], idx_plus_len, axis=0,
                                   keepdims=False)
    scalars = jnp.broadcast_to(
        jnp.stack([alpha.astype(jnp.float32),
                   jnp.asarray(x_scale, jnp.float32)])[:, None], (2, _L))

    out = _sc_kernel_call(emb_table, y32, scalars, pos)
    return out.reshape(BATCH, Q_LEN, D_MODEL)


# speedup vs baseline: 1.1726x; 1.1726x over previous
import functools

import jax
import jax.numpy as jnp
from jax import lax
from jax.experimental import pallas as pl
from jax.experimental.pallas import tpu as pltpu
from jax.experimental.pallas import tpu_sc as plsc

VOCAB = 100000
D_MODEL = 1024
BATCH = 128
Q_LEN = 32

_L = 16
_NVEC = D_MODEL // _L
_B = BATCH * Q_LEN
_NIN = 5
_NOUT = 2
_CH = 16
_NCH = 8
_CBLK = 8
_BROWS = BATCH // 32


def _sc_kernel_call(table, y, scalars, pos):
    info = plsc.get_sparse_core_info()
    nc, ns = info.num_cores, info.num_subcores
    nw = nc * ns
    rows_per_w = _B // nw
    assert rows_per_w == _NCH * _CH

    mesh = plsc.VectorSubcoreMesh(core_axis_name="c", subcore_axis_name="s")

    @functools.partial(
        pl.kernel,
        mesh=mesh,
        out_type=jax.ShapeDtypeStruct((_B, D_MODEL), jnp.float32),
        scratch_types=(
            [pltpu.VMEM((_BROWS, Q_LEN), jnp.int32)]
            + [pltpu.VMEM((_CH, D_MODEL), jnp.float32)
               for _ in range(_NIN + _NOUT)]
            + [pltpu.VMEM((D_MODEL,), jnp.float32),
               pltpu.VMEM((D_MODEL,), jnp.float32),
               pltpu.VMEM((2, _L), jnp.float32)]
            + [pltpu.SemaphoreType.DMA for _ in range(_NIN + _NOUT + 1)]
        ),
    )
    def k(table_hbm, y_hbm, sc_hbm, pos_hbm, out_hbm, *refs):
        idx_v = refs[0]
        ibufs = refs[1:1 + _NIN]
        obufs = refs[1 + _NIN:1 + _NIN + _NOUT]
        pos_v, spos, scal_v = refs[1 + _NIN + _NOUT:4 + _NIN + _NOUT]
        gsems = refs[4 + _NIN + _NOUT:4 + 2 * _NIN + _NOUT]
        wsems = refs[4 + 2 * _NIN + _NOUT:4 + 2 * _NIN + 2 * _NOUT]
        psem = refs[4 + 2 * _NIN + 2 * _NOUT]

        wid = lax.axis_index("s") * nc + lax.axis_index("c")
        base = wid * rows_per_w

        scal_cp = pltpu.async_copy(sc_hbm, scal_v, psem)
        pos_cp = pltpu.async_copy(pos_hbm, pos_v, psem)
        pltpu.sync_copy(y_hbm.at[pl.ds(wid * _BROWS, _BROWS)], idx_v)

        def start_gather(c, s):
            row, off = (c * _CH) // Q_LEN, (c * _CH) % Q_LEN
            idx = idx_v.at[row, pl.ds(off, _CH)]
            return pltpu.async_copy(table_hbm.at[idx], ibufs[s], gsems[s])

        def start_write(c, s):
            return pltpu.async_copy(obufs[s],
                                    out_hbm.at[pl.ds(base + c * _CH, _CH)],
                                    wsems[s])

        inflight_g = [None] * _NIN
        for c in range(_NIN - 1):
            inflight_g[c] = start_gather(c, c)

        scal_cp.wait()
        pos_cp.wait()

        av = scal_v[0, :]
        sv = scal_v[1, :]

        def scale_pos(j, carry):
            spos[pl.ds(j * _L, _L)] = pos_v[pl.ds(j * _L, _L)] * av
            return carry

        lax.fori_loop(0, _NVEC, scale_pos, 0)

        def compute(si, so):
            src = ibufs[si]
            dst = obufs[so]

            def block_body(b, carry):
                pvs = [spos[pl.ds((b * _CBLK + j) * _L, _L)]
                       for j in range(_CBLK)]

                @plsc.parallel_loop(0, _CH, 1, unroll=2)
                def rows_body(r):
                    for j in range(_CBLK):
                        sl = pl.ds((b * _CBLK + j) * _L, _L)
                        dst[r, sl] = src[r, sl] * sv + pvs[j]

                return carry

            lax.fori_loop(0, _NVEC // _CBLK, block_body, 0)

        inflight_w = [None] * _NOUT
        for i in range(_NCH):
            si = i % _NIN
            so = i % _NOUT
            j = i + _NIN - 1
            if j < _NCH:
                inflight_g[j % _NIN] = start_gather(j, j % _NIN)
            inflight_g[si].wait()
            if inflight_w[so] is not None:
                inflight_w[so].wait()
                inflight_w[so] = None
            compute(si, so)
            inflight_w[so] = start_write(i, so)
        for s in range(_NOUT):
            if inflight_w[s] is not None:
                inflight_w[s].wait()

    return k(table, y, scalars, pos)


def kernel(emb_table, alpha, pe, x_scale, y, idx_plus_len):
    y32 = y.astype(jnp.int32)
    pos = lax.dynamic_index_in_dim(pe[0], idx_plus_len, axis=0,
                                   keepdims=False)
    scalars = jnp.broadcast_to(
        jnp.stack([alpha.astype(jnp.float32),
                   jnp.asarray(x_scale, jnp.float32)])[:, None], (2, _L))

    out = _sc_kernel_call(emb_table, y32, scalars, pos)
    return out.reshape(BATCH, Q_LEN, D_MODEL)
